# baseline (speedup 1.0000x reference)
import functools
import math

import jax
import jax.numpy as jnp
from jax import lax
from jax.experimental import pallas as pl
from jax.experimental.pallas import tpu as pltpu
from jax.experimental.pallas import tpu_sc as plsc

D_MODEL = 64
_SCALE = 1.0 / math.sqrt(D_MODEL)
NC = 2
NS = 16
NW = NC * NS
LANES = 16
CH = 128


@functools.lru_cache(maxsize=None)
def _build(nch):
    mesh = plsc.VectorSubcoreMesh(core_axis_name="c", subcore_axis_name="s")

    @functools.partial(
        pl.kernel,
        mesh=mesh,
        compiler_params=pltpu.CompilerParams(use_tc_tiling_on_sc=False),
        out_type=jax.ShapeDtypeStruct((NW * nch * CH, D_MODEL), jnp.float32),
        scratch_types=[
            pltpu.VMEM((nch, CH), jnp.int32),
            pltpu.VMEM((CH, D_MODEL), jnp.float32),
            pltpu.SemaphoreType.DMA,
        ],
    )
    def emb(idx_hbm, table_hbm, out_hbm, idx_v, rows_v, sem):
        wid = lax.axis_index("s") * NC + lax.axis_index("c")
        pltpu.sync_copy(idx_hbm.at[wid], idx_v)

        def chunk(j, carry):
            pltpu.async_copy(table_hbm.at[idx_v.at[j]], rows_v, sem).wait()

            def scale(i, c):
                r = i // (D_MODEL // LANES)
                col = (i % (D_MODEL // LANES)) * LANES
                rows_v[r, pl.ds(col, LANES)] = (
                    rows_v[r, pl.ds(col, LANES)] * _SCALE
                )
                return c

            lax.fori_loop(0, CH * (D_MODEL // LANES), scale, 0)
            pltpu.sync_copy(rows_v, out_hbm.at[pl.ds((wid * nch + j) * CH, CH)])
            return carry

        lax.fori_loop(0, nch, chunk, 0)

    return emb


def kernel(x, table):
    b, l = x.shape
    bt = b * l
    nch = bt // (NW * CH)
    idx = x.reshape(NW, nch, CH)
    out = _build(nch)(idx, table)
    return out.reshape(b, l, D_MODEL)


# speedup vs baseline: 1.5857x; 1.5857x over previous
import functools
import math

import jax
import jax.numpy as jnp
from jax import lax
from jax.experimental import pallas as pl
from jax.experimental.pallas import tpu as pltpu
from jax.experimental.pallas import tpu_sc as plsc

D_MODEL = 64
_SCALE = 1.0 / math.sqrt(D_MODEL)
NC = 2
NS = 16
NW = NC * NS
LANES = 16
CH = 128
NBUF = 4


@functools.lru_cache(maxsize=None)
def _build(nch):
    assert nch % NBUF == 0
    mesh = plsc.VectorSubcoreMesh(core_axis_name="c", subcore_axis_name="s")

    @functools.partial(
        pl.kernel,
        mesh=mesh,
        compiler_params=pltpu.CompilerParams(use_tc_tiling_on_sc=False),
        out_type=jax.ShapeDtypeStruct((NW * nch * CH, D_MODEL), jnp.float32),
        scratch_types=[
            pltpu.VMEM((nch, CH), jnp.int32),
        ]
        + [pltpu.VMEM((CH, D_MODEL), jnp.float32) for _ in range(NBUF)]
        + [pltpu.SemaphoreType.DMA for _ in range(2 * NBUF)],
    )
    def emb(idx_hbm, table_hbm, out_hbm, idx_v, *rest):
        bufs = rest[:NBUF]
        gsems = rest[NBUF:2 * NBUF]
        ssems = rest[2 * NBUF:]
        wid = lax.axis_index("s") * NC + lax.axis_index("c")
        base = wid * nch
        pltpu.sync_copy(idx_hbm.at[wid], idx_v)

        def gather(jn, b):
            pltpu.async_copy(table_hbm.at[idx_v.at[jn]], bufs[b], gsems[b])

        def wait_gather(b):
            pltpu.make_async_copy(
                table_hbm.at[idx_v.at[0]], bufs[b], gsems[b]
            ).wait()

        def store(j, b):
            pltpu.async_copy(
                bufs[b], out_hbm.at[pl.ds((base + j) * CH, CH)], ssems[b]
            )

        def wait_store(b):
            pltpu.make_async_copy(
                bufs[b], out_hbm.at[pl.ds(0, CH)], ssems[b]
            ).wait()

        def refill(b, jn):
            @pl.when(jn < nch)
            def _():
                wait_store(b)
                gather(jn, b)

        def scale(buf):
            @plsc.parallel_loop(0, CH, unroll=8)
            def _(r):
                for col in range(0, D_MODEL, LANES):
                    buf[r, pl.ds(col, LANES)] = (
                        buf[r, pl.ds(col, LANES)] * _SCALE
                    )

        for b in range(NBUF):
            gather(b, b)

        @pl.loop(0, nch // NBUF)
        def _(k):
            j0 = k * NBUF
            for b in range(NBUF):
                wait_gather(b)
                scale(bufs[b])
                store(j0 + b, b)
                if b >= 1:
                    refill(b - 1, j0 + NBUF + b - 1)
            refill(NBUF - 1, j0 + 2 * NBUF - 1)

        for b in range(NBUF):
            wait_store(b)

    return emb


def kernel(x, table):
    b, l = x.shape
    bt = b * l
    nch = bt // (NW * CH)
    idx = x.reshape(NW, nch, CH)
    out = _build(nch)(idx, table)
    return out.reshape(b, l, D_MODEL)
